# Initial kernel scaffold; baseline (speedup 1.0000x reference)
import jax
import jax.numpy as jnp
from jax.experimental import pallas as pl


def kernel(x, emb, W1, b1, W2, b2, Wf, bf):
    raise NotImplementedError("write your pallas kernel here")


# speedup vs baseline: 6.9686x; 6.9686x over previous
import functools

import jax
import jax.numpy as jnp
from jax import lax
from jax.experimental import pallas as pl
from jax.experimental.pallas import tpu as pltpu
from jax.experimental.pallas import tpu_sc as plsc

VOCAB = 100000
EMB = 128
HID = 1024
TAGS = 1000
B = 4096
L = 200

NC = 2
NS = 16
NW = NC * NS
BPW = B // NW
LANES = 16
NCH = EMB // LANES

L0 = 128
L1 = L - L0


def _pool_body(x_hbm, emb_hbm, out_hbm, idx_v, rows_v, acc_v, sem0, sem1):
    wid = lax.axis_index("s") * NC + lax.axis_index("c")
    base = wid * BPW

    @pl.loop(0, BPW)
    def _row(i):
        r = base + i
        pltpu.sync_copy(x_hbm.at[r], idx_v)
        cp0 = pltpu.async_copy(
            emb_hbm.at[idx_v.at[pl.ds(0, L0)]], rows_v.at[pl.ds(0, L0)], sem0)
        cp1 = pltpu.async_copy(
            emb_hbm.at[idx_v.at[pl.ds(L0, L1)]], rows_v.at[pl.ds(L0, L1)], sem1)
        cp0.wait()
        cp1.wait()

        zeros = tuple(jnp.zeros((LANES,), jnp.float32) for _ in range(NCH))

        @pl.loop(0, L, init_carry=zeros, unroll=2)
        def _acc(l, carry):
            return tuple(
                c + rows_v[l, pl.ds(j * LANES, LANES)]
                for j, c in enumerate(carry))

        for j in range(NCH):
            acc_v[pl.ds(j * LANES, LANES)] = _acc[j]
        pltpu.sync_copy(acc_v, out_hbm.at[r])


_pool = pl.kernel(
    _pool_body,
    out_type=jax.ShapeDtypeStruct((B, EMB), jnp.float32),
    mesh=plsc.VectorSubcoreMesh(core_axis_name="c", subcore_axis_name="s"),
    scratch_types=[
        pltpu.VMEM((L,), jnp.int32),
        pltpu.VMEM((L, EMB), jnp.float32),
        pltpu.VMEM((EMB,), jnp.float32),
        pltpu.SemaphoreType.DMA,
        pltpu.SemaphoreType.DMA,
    ],
)


BB = 512


def _mlp_body(x_ref, p_ref, w1_ref, b1_ref, w2_ref, b2_ref, wf_ref, bf_ref,
              o_ref):
    cnt = jnp.sum((x_ref[...] != 0).astype(jnp.float32), axis=1, keepdims=True)
    h = p_ref[...] / (cnt + 1e-05)
    h = lax.dot_general(h, w1_ref[...], (((1,), (1,)), ((), ())),
                        preferred_element_type=jnp.float32) + b1_ref[...]
    h = jnp.where(h > 0, h, 0.2 * h)
    h = lax.dot_general(h, w2_ref[...], (((1,), (1,)), ((), ())),
                        preferred_element_type=jnp.float32) + b2_ref[...]
    h = jnp.where(h > 0, h, 0.2 * h)
    o_ref[...] = lax.dot_general(h, wf_ref[...], (((1,), (1,)), ((), ())),
                                 preferred_element_type=jnp.float32) + bf_ref[...]


def _mlp(pooled, x, W1, b1, W2, b2, Wf, bf):
    grid = (B // BB,)
    return pl.pallas_call(
        _mlp_body,
        grid=grid,
        in_specs=[
            pl.BlockSpec((BB, L), lambda i: (i, 0)),
            pl.BlockSpec((BB, EMB), lambda i: (i, 0)),
            pl.BlockSpec((HID, EMB), lambda i: (0, 0)),
            pl.BlockSpec((1, HID), lambda i: (0, 0)),
            pl.BlockSpec((HID, HID), lambda i: (0, 0)),
            pl.BlockSpec((1, HID), lambda i: (0, 0)),
            pl.BlockSpec((TAGS, HID), lambda i: (0, 0)),
            pl.BlockSpec((1, TAGS), lambda i: (0, 0)),
        ],
        out_specs=pl.BlockSpec((BB, TAGS), lambda i: (i, 0)),
        out_shape=jax.ShapeDtypeStruct((B, TAGS), jnp.float32),
    )(x, pooled, W1, b1, W2, b2, Wf, bf)


@jax.jit
def kernel(x, emb, W1, b1, W2, b2, Wf, bf):
    pooled = _pool(x, emb)
    return _mlp(pooled, x, W1, b1[None, :], W2, b2[None, :], Wf, bf[None, :])


# speedup vs baseline: 10.9607x; 1.5729x over previous
import functools

import jax
import jax.numpy as jnp
from jax import lax
from jax.experimental import pallas as pl
from jax.experimental.pallas import tpu as pltpu
from jax.experimental.pallas import tpu_sc as plsc

VOCAB = 100000
EMB = 128
HID = 1024
TAGS = 1000
B = 4096
L = 200

NC = 2
NS = 16
NW = NC * NS
BPW = B // NW
LANES = 16
NCH = EMB // LANES

L0 = 128
L1 = L - L0


def _pool_body(x_hbm, emb_hbm, out_hbm,
               idx0, idx1, rows0, rows1, acc0, acc1,
               isem0, isem1, gsem0, gsem1, osem0, osem1):
    idx = [idx0, idx1]
    rows = [rows0, rows1]
    acc = [acc0, acc1]
    isem = [isem0, isem1]
    gsem = [gsem0, gsem1]
    osem = [osem0, osem1]
    wid = lax.axis_index("s") * NC + lax.axis_index("c")
    base = wid * BPW

    def fire_gather(s):
        pltpu.async_copy(
            emb_hbm.at[idx[s].at[pl.ds(0, L0)]], rows[s].at[pl.ds(0, L0)],
            gsem[s])
        pltpu.async_copy(
            emb_hbm.at[idx[s].at[pl.ds(L0, L1)]], rows[s].at[pl.ds(L0, L1)],
            gsem[s])

    def wait_gather(s):
        pltpu.make_async_copy(emb_hbm.at[pl.ds(0, L)], rows[s], gsem[s]).wait()

    pltpu.sync_copy(x_hbm.at[base], idx[0])
    fire_gather(0)
    pltpu.async_copy(x_hbm.at[base + 1], idx[1], isem[1])

    @pl.loop(0, BPW, step=2)
    def _outer(io):
        for s in range(2):
            i = io + s
            cur, nxt = s, 1 - s
            wait_gather(cur)

            @pl.when(i + 1 < BPW)
            def _():
                pltpu.make_async_copy(
                    x_hbm.at[base + i + 1], idx[nxt], isem[nxt]).wait()
                fire_gather(nxt)

            @pl.when(i + 2 < BPW)
            def _():
                pltpu.async_copy(x_hbm.at[base + i + 2], idx[cur], isem[cur])

            @pl.when(i >= 2)
            def _():
                pltpu.make_async_copy(
                    acc[cur], out_hbm.at[base + i - 2], osem[cur]).wait()

            zeros = tuple(jnp.zeros((LANES,), jnp.float32) for _ in range(NCH))

            @pl.loop(0, L, init_carry=zeros, unroll=2)
            def _acc(l, carry):
                return tuple(
                    c + rows[cur][l, pl.ds(j * LANES, LANES)]
                    for j, c in enumerate(carry))

            for j in range(NCH):
                acc[cur][pl.ds(j * LANES, LANES)] = _acc[j]
            pltpu.async_copy(acc[cur], out_hbm.at[base + i], osem[cur])

    pltpu.make_async_copy(acc[0], out_hbm.at[base + BPW - 2], osem[0]).wait()
    pltpu.make_async_copy(acc[1], out_hbm.at[base + BPW - 1], osem[1]).wait()


_pool = pl.kernel(
    _pool_body,
    out_type=jax.ShapeDtypeStruct((B, EMB), jnp.float32),
    mesh=plsc.VectorSubcoreMesh(core_axis_name="c", subcore_axis_name="s"),
    scratch_types=(
        [pltpu.VMEM((L,), jnp.int32)] * 2
        + [pltpu.VMEM((L, EMB), jnp.float32)] * 2
        + [pltpu.VMEM((EMB,), jnp.float32)] * 2
        + [pltpu.SemaphoreType.DMA] * 6
    ),
)


BB = 512


def _mlp_body(x_ref, p_ref, w1_ref, b1_ref, w2_ref, b2_ref, wf_ref, bf_ref,
              o_ref):
    cnt = jnp.sum((x_ref[...] != 0).astype(jnp.float32), axis=1, keepdims=True)
    h = p_ref[...] / (cnt + 1e-05)
    h = lax.dot_general(h, w1_ref[...], (((1,), (1,)), ((), ())),
                        preferred_element_type=jnp.float32) + b1_ref[...]
    h = jnp.where(h > 0, h, 0.2 * h)
    h = lax.dot_general(h, w2_ref[...], (((1,), (1,)), ((), ())),
                        preferred_element_type=jnp.float32) + b2_ref[...]
    h = jnp.where(h > 0, h, 0.2 * h)
    o_ref[...] = lax.dot_general(h, wf_ref[...], (((1,), (1,)), ((), ())),
                                 preferred_element_type=jnp.float32) + bf_ref[...]


def _mlp(pooled, x, W1, b1, W2, b2, Wf, bf):
    grid = (B // BB,)
    return pl.pallas_call(
        _mlp_body,
        grid=grid,
        in_specs=[
            pl.BlockSpec((BB, L), lambda i: (i, 0)),
            pl.BlockSpec((BB, EMB), lambda i: (i, 0)),
            pl.BlockSpec((HID, EMB), lambda i: (0, 0)),
            pl.BlockSpec((1, HID), lambda i: (0, 0)),
            pl.BlockSpec((HID, HID), lambda i: (0, 0)),
            pl.BlockSpec((1, HID), lambda i: (0, 0)),
            pl.BlockSpec((TAGS, HID), lambda i: (0, 0)),
            pl.BlockSpec((1, TAGS), lambda i: (0, 0)),
        ],
        out_specs=pl.BlockSpec((BB, TAGS), lambda i: (i, 0)),
        out_shape=jax.ShapeDtypeStruct((B, TAGS), jnp.float32),
    )(x, pooled, W1, b1, W2, b2, Wf, bf)


@jax.jit
def kernel(x, emb, W1, b1, W2, b2, Wf, bf):
    pooled = _pool(x, emb)
    return _mlp(pooled, x, W1, b1[None, :], W2, b2[None, :], Wf, bf[None, :])


# speedup vs baseline: 15.6011x; 1.4234x over previous
import functools

import numpy as np
import jax
import jax.numpy as jnp
from jax import lax
from jax.experimental import pallas as pl
from jax.experimental.pallas import tpu as pltpu
from jax.experimental.pallas import tpu_sc as plsc

VOCAB = 100000
EMB = 128
HID = 1024
TAGS = 1000
B = 4096
L = 200

NC = 2
NS = 16
NW = NC * NS
BPW = B // NW
LANES = 16
NCH = EMB // LANES

L0 = 128
L1 = L - L0


NBUF = 4


def _pool_body(x_hbm, emb_hbm, out_hbm, *refs):
    idx = list(refs[0:NBUF])
    rows = list(refs[NBUF:2 * NBUF])
    acc = list(refs[2 * NBUF:3 * NBUF])
    isem = list(refs[3 * NBUF:4 * NBUF])
    gsem = list(refs[4 * NBUF:5 * NBUF])
    osem = list(refs[5 * NBUF:6 * NBUF])
    wid = lax.axis_index("s") * NC + lax.axis_index("c")
    base = wid * BPW

    def fire_gather(s):
        pltpu.async_copy(
            emb_hbm.at[idx[s].at[pl.ds(0, L0)]], rows[s].at[pl.ds(0, L0)],
            gsem[s])
        pltpu.async_copy(
            emb_hbm.at[idx[s].at[pl.ds(L0, L1)]], rows[s].at[pl.ds(L0, L1)],
            gsem[s])

    def wait_gather(s):
        pltpu.make_async_copy(emb_hbm.at[pl.ds(0, L)], rows[s], gsem[s]).wait()

    for s in range(NBUF - 1):
        pltpu.sync_copy(x_hbm.at[base + s], idx[s])
        fire_gather(s)
    pltpu.async_copy(x_hbm.at[base + NBUF - 1], idx[NBUF - 1], isem[NBUF - 1])

    @pl.loop(0, BPW, step=NBUF)
    def _outer(io):
        for s in range(NBUF):
            i = io + s
            nxt = (s + NBUF - 1) % NBUF
            wait_gather(s)

            @pl.when(i + NBUF - 1 < BPW)
            def _():
                pltpu.make_async_copy(
                    x_hbm.at[base + i + NBUF - 1], idx[nxt], isem[nxt]).wait()
                fire_gather(nxt)

            @pl.when(i + NBUF < BPW)
            def _():
                pltpu.async_copy(x_hbm.at[base + i + NBUF], idx[s], isem[s])

            @pl.when(i >= NBUF)
            def _():
                pltpu.make_async_copy(
                    acc[s], out_hbm.at[base + i - NBUF], osem[s]).wait()

            zeros = tuple(jnp.zeros((LANES,), jnp.float32) for _ in range(NCH))

            @pl.loop(0, L, init_carry=zeros, unroll=2)
            def _acc(l, carry):
                return tuple(
                    c + rows[s][l, pl.ds(j * LANES, LANES)]
                    for j, c in enumerate(carry))

            for j in range(NCH):
                acc[s][pl.ds(j * LANES, LANES)] = _acc[j]
            pltpu.async_copy(acc[s], out_hbm.at[base + i], osem[s])

    for s in range(NBUF):
        pltpu.make_async_copy(
            acc[s], out_hbm.at[base + BPW - NBUF + s], osem[s]).wait()


_pool = pl.kernel(
    _pool_body,
    out_type=jax.ShapeDtypeStruct((B, EMB), jnp.float32),
    mesh=plsc.VectorSubcoreMesh(core_axis_name="c", subcore_axis_name="s"),
    scratch_types=(
        [pltpu.VMEM((L,), jnp.int32)] * NBUF
        + [pltpu.VMEM((L, EMB), jnp.float32)] * NBUF
        + [pltpu.VMEM((EMB,), jnp.float32)] * NBUF
        + [pltpu.SemaphoreType.DMA] * (3 * NBUF)
    ),
)


BB = 512


def _mlp_body(x_ref, p_ref, w1_ref, b1_ref, w2_ref, b2_ref, wf_ref, bf_ref,
              o_ref):
    cnt = jnp.sum((x_ref[...] != 0).astype(jnp.float32), axis=1, keepdims=True)
    h = p_ref[...] / (cnt + 1e-05)
    h = lax.dot_general(h, w1_ref[...], (((1,), (1,)), ((), ())),
                        preferred_element_type=jnp.float32) + b1_ref[...]
    h = jnp.where(h > 0, h, 0.2 * h)
    h = lax.dot_general(h, w2_ref[...], (((1,), (1,)), ((), ())),
                        preferred_element_type=jnp.float32) + b2_ref[...]
    h = jnp.where(h > 0, h, 0.2 * h)
    o_ref[...] = lax.dot_general(h, wf_ref[...], (((1,), (1,)), ((), ())),
                                 preferred_element_type=jnp.float32) + bf_ref[...]


def _mlp(pooled, x, W1, b1, W2, b2, Wf, bf):
    grid = (B // BB,)
    return pl.pallas_call(
        _mlp_body,
        grid=grid,
        in_specs=[
            pl.BlockSpec((BB, L), lambda i: (i, 0)),
            pl.BlockSpec((BB, EMB), lambda i: (i, 0)),
            pl.BlockSpec((HID, EMB), lambda i: (0, 0)),
            pl.BlockSpec((1, HID), lambda i: (0, 0)),
            pl.BlockSpec((HID, HID), lambda i: (0, 0)),
            pl.BlockSpec((1, HID), lambda i: (0, 0)),
            pl.BlockSpec((TAGS, HID), lambda i: (0, 0)),
            pl.BlockSpec((1, TAGS), lambda i: (0, 0)),
        ],
        out_specs=pl.BlockSpec((BB, TAGS), lambda i: (i, 0)),
        out_shape=jax.ShapeDtypeStruct((B, TAGS), jnp.float32),
    )(x, pooled, W1, b1, W2, b2, Wf, bf)


@jax.jit
def kernel(x, emb, W1, b1, W2, b2, Wf, bf):
    pooled = _pool(x, emb)
    return _mlp(pooled, x, W1, b1[None, :], W2, b2[None, :], Wf, bf[None, :])
